# baseline (speedup 1.0000x reference)
import math

import jax
import jax.numpy as jnp
from jax.experimental import pallas as pl
from jax.experimental.pallas import tpu as pltpu

B = 32
L = 512
NV = 7
D = 256
PLEN = 16
STRIDE = 8
PADDING = 8
PRED = 96
TOPK = 5
MEM = 100000
NH = 4
HD = D // NH
NP = (L + PADDING - PLEN) // STRIDE + 1
ROWS = B * NV
TILE = 2000
NT = MEM // TILE
NEG = -1e30


def _gelu(x):
    return 0.5 * x * (1.0 + jax.lax.erf(x * 0.7071067811865476))


def _k0(pf_ref, w_ref, b_ref, emb_ref, q_ref):
    e = jax.lax.dot_general(
        pf_ref[...].astype(jnp.bfloat16), w_ref[...].astype(jnp.bfloat16),
        dimension_numbers=(((1,), (0,)), ((), ())),
        precision=jax.lax.Precision.DEFAULT,
        preferred_element_type=jnp.float32) + b_ref[...]
    emb_ref[...] = e
    nq = e.shape[0] // NP
    q_ref[0] = jnp.mean(e.reshape(nq, NP, D), axis=1)


def _k1(q_ref, mem_ref, idx_out_ref, rv_ref, ri_ref):
    t = pl.program_id(0)

    @pl.when(t == 0)
    def _():
        rv_ref[...] = jnp.full((ROWS, TOPK), NEG, jnp.float32)
        ri_ref[...] = jnp.zeros((ROWS, TOPK), jnp.int32)

    sim = jax.lax.dot_general(
        q_ref[...].astype(jnp.bfloat16), mem_ref[...].astype(jnp.bfloat16),
        dimension_numbers=(((1,), (1,)), ((), ())),
        precision=jax.lax.Precision.DEFAULT,
        preferred_element_type=jnp.float32)
    base = t * TILE
    iota_t = jax.lax.broadcasted_iota(jnp.int32, (ROWS, TILE), 1) + base
    aug = jnp.concatenate([sim, rv_ref[...]], axis=1)
    aug_idx = jnp.concatenate([iota_t, ri_ref[...]], axis=1)
    iota_a = jax.lax.broadcasted_iota(jnp.int32, (ROWS, TILE + TOPK), 1)

    new_v = []
    new_i = []
    for _j in range(TOPK):
        m = jnp.max(aug, axis=1, keepdims=True)
        pos = jnp.min(jnp.where(aug == m, iota_a, TILE + TOPK),
                      axis=1, keepdims=True)
        sel = iota_a == pos
        new_v.append(m)
        new_i.append(jnp.sum(jnp.where(sel, aug_idx, 0), axis=1,
                             keepdims=True))
        aug = jnp.where(sel, NEG, aug)
    rv_ref[...] = jnp.concatenate(new_v, axis=1)
    ri_ref[...] = jnp.concatenate(new_i, axis=1)

    @pl.when(t == NT - 1)
    def _():
        idx_out_ref[...] = ri_ref[...]


def _k2(idx_ref, r0, r1, r2, r3, r4, w1_ref, b1_ref, w2_ref, b2_ref,
        out_ref):
    rows = jnp.concatenate(
        [r0[0], r1[0], r2[0], r3[0], r4[0]], axis=0)
    h = _gelu(rows @ w1_ref[...] + b1_ref[...])
    lm = h @ w2_ref[...] + b2_ref[...]
    out_ref[0] = jnp.mean(lm, axis=0, keepdims=True)


def _k3(x_ref, wq_ref, bq_ref, wk_ref, bk_ref, wv_ref, bv_ref,
        wo_ref, bo_ref, out_ref):
    nb = x_ref.shape[0]
    x2 = x_ref[...].reshape(nb * NP, D)
    q = x2 @ wq_ref[...] + bq_ref[...]
    k = x2 @ wk_ref[...] + bk_ref[...]
    v = x2 @ wv_ref[...] + bv_ref[...]
    scale = 1.0 / math.sqrt(HD)
    outs = []
    for h in range(NH):
        sl = slice(h * HD, (h + 1) * HD)
        qh = q[:, sl].reshape(nb, NP, HD)
        kh = k[:, sl].reshape(nb, NP, HD)
        vh = v[:, sl].reshape(nb, NP, HD)
        s = jax.lax.dot_general(
            qh, kh, dimension_numbers=(((2,), (2,)), ((0,), (0,))),
            preferred_element_type=jnp.float32) * scale
        m = jnp.max(s, axis=-1, keepdims=True)
        e = jnp.exp(s - m)
        a = e / jnp.sum(e, axis=-1, keepdims=True)
        oh = jax.lax.dot_general(
            a, vh, dimension_numbers=(((2,), (1,)), ((0,), (0,))),
            preferred_element_type=jnp.float32)
        outs.append(oh.reshape(nb * NP, HD))
    o = jnp.concatenate(outs, axis=1)
    out_ref[...] = (o @ wo_ref[...] + bo_ref[...]).reshape(nb, NP, D)


def _k4(emb_ref, lm_ref, glob_ref, wg1a_ref, wg1b_ref, bg1_ref,
        wg2_ref, bg2_ref, wmh_ref, bmh_ref, wf1_ref, bf1_ref,
        wf2_ref, bf2_ref, out_ref, acc_ref):
    p = pl.program_id(0)
    loc = emb_ref[0] + lm_ref[...]
    glo = glob_ref[0]
    z = _gelu(loc @ wg1a_ref[...] + glo @ wg1b_ref[...] + bg1_ref[...])
    s2 = z @ wg2_ref[...] + bg2_ref[...]
    g0 = jax.nn.sigmoid(s2[:, 0:1] - s2[:, 1:2])
    g1 = 1.0 - g0
    mf = g0 * loc + g1 * glo
    contrib = mf @ wmh_ref[0]

    @pl.when(p == 0)
    def _():
        acc_ref[...] = contrib

    @pl.when(p > 0)
    def _():
        acc_ref[...] = acc_ref[...] + contrib

    @pl.when(p == NP - 1)
    def _():
        mh = acc_ref[...] + bmh_ref[...]
        f = _gelu(_gelu(mh @ wf1_ref[...] + bf1_ref[...])
                  @ wf2_ref[...] + bf2_ref[...])
        out_ref[...] = f + mh


def kernel(x_enc, W_pe, b_pe, mem_bank, W_lm1, b_lm1, W_lm2, b_lm2,
           W_q, b_q, W_k, b_k, W_v, b_v, W_o, b_o,
           W_g1, b_g1, W_g2, b_g2, W_mh, b_mh, W_f1, b_f1, W_f2, b_f2):
    with jax.default_matmul_precision('highest'):
        return _impl(x_enc, W_pe, b_pe, mem_bank, W_lm1, b_lm1, W_lm2,
                     b_lm2, W_q, b_q, W_k, b_k, W_v, b_v, W_o, b_o,
                     W_g1, b_g1, W_g2, b_g2, W_mh, b_mh, W_f1, b_f1,
                     W_f2, b_f2)


def _impl(x_enc, W_pe, b_pe, mem_bank, W_lm1, b_lm1, W_lm2, b_lm2,
          W_q, b_q, W_k, b_k, W_v, b_v, W_o, b_o,
          W_g1, b_g1, W_g2, b_g2, W_mh, b_mh, W_f1, b_f1, W_f2, b_f2):
    f32 = jnp.float32
    means = jnp.mean(x_enc, axis=1, keepdims=True)
    xc = x_enc - means
    stdev = jnp.sqrt(jnp.var(xc, axis=1, keepdims=True) + 1e-05)
    xn = xc / stdev
    xt = xn.transpose(0, 2, 1)
    xp = jnp.concatenate([xt, jnp.repeat(xt[:, :, -1:], PADDING, axis=-1)],
                         axis=-1)
    uidx = jnp.arange(NP)[:, None] * STRIDE + jnp.arange(PLEN)[None, :]
    patches = xp[:, :, uidx].reshape(ROWS, NP, PLEN)
    pflat = patches.reshape(ROWS * NP, PLEN)
    pmean = patches.mean(axis=1)

    r2 = lambda b: b.reshape(1, -1)

    NB0 = 16
    RB = ROWS * NP // NB0
    emb_flat, q3 = pl.pallas_call(
        _k0,
        grid=(NB0,),
        in_specs=[pl.BlockSpec((RB, PLEN), lambda i: (i, 0)),
                  pl.BlockSpec((PLEN, D), lambda i: (0, 0)),
                  pl.BlockSpec((1, D), lambda i: (0, 0))],
        out_specs=[pl.BlockSpec((RB, D), lambda i: (i, 0)),
                   pl.BlockSpec((1, RB // NP, D), lambda i: (i, 0, 0))],
        out_shape=[jax.ShapeDtypeStruct((ROWS * NP, D), f32),
                   jax.ShapeDtypeStruct((NB0, RB // NP, D), f32)],
    )(pflat, W_pe, r2(b_pe))
    emb3 = emb_flat.reshape(ROWS, NP, D)
    query = q3.reshape(ROWS, D)

    topk_idx = pl.pallas_call(
        _k1,
        grid=(NT,),
        in_specs=[
            pl.BlockSpec((ROWS, D), lambda t: (0, 0)),
            pl.BlockSpec((TILE, D), lambda t: (t, 0)),
        ],
        out_specs=pl.BlockSpec((ROWS, TOPK), lambda t: (0, 0)),
        out_shape=jax.ShapeDtypeStruct((ROWS, TOPK), jnp.int32),
        scratch_shapes=[pltpu.VMEM((ROWS, TOPK), f32),
                        pltpu.VMEM((ROWS, TOPK), jnp.int32)],
    )(query, mem_bank)
    idx_flat = topk_idx.reshape(-1)

    mem3 = mem_bank.reshape(MEM, 1, D)
    row_spec = [
        pl.BlockSpec((1, 1, D), (lambda i, idx, j=j: (idx[i * TOPK + j], 0, 0)))
        for j in range(TOPK)
    ]
    lm_mean = pl.pallas_call(
        _k2,
        grid_spec=pltpu.PrefetchScalarGridSpec(
            num_scalar_prefetch=1,
            grid=(ROWS,),
            in_specs=row_spec + [
                pl.BlockSpec((D, 2 * D), lambda i, idx: (0, 0)),
                pl.BlockSpec((1, 2 * D), lambda i, idx: (0, 0)),
                pl.BlockSpec((2 * D, D), lambda i, idx: (0, 0)),
                pl.BlockSpec((1, D), lambda i, idx: (0, 0)),
            ],
            out_specs=pl.BlockSpec((1, 1, D), lambda i, idx: (i, 0, 0)),
        ),
        out_shape=jax.ShapeDtypeStruct((ROWS, 1, D), f32),
    )(idx_flat, mem3, mem3, mem3, mem3, mem3,
      W_lm1, r2(b_lm1), W_lm2, r2(b_lm2))
    lm_mean = lm_mean.reshape(ROWS, D)

    BB = 16
    glob = pl.pallas_call(
        _k3,
        grid=(ROWS // BB,),
        in_specs=[pl.BlockSpec((BB, NP, D), lambda i: (i, 0, 0))] + [
            pl.BlockSpec(s, lambda i: (0, 0))
            for s in [(D, D), (1, D)] * 4
        ],
        out_specs=pl.BlockSpec((BB, NP, D), lambda i: (i, 0, 0)),
        out_shape=jax.ShapeDtypeStruct((ROWS, NP, D), f32),
    )(emb3, W_q, r2(b_q), W_k, r2(b_k), W_v, r2(b_v), W_o, r2(b_o))

    wmh3 = W_mh.reshape(NP, D, PRED)
    preds = pl.pallas_call(
        _k4,
        grid=(NP,),
        in_specs=[
            pl.BlockSpec((1, ROWS, D), lambda p: (p, 0, 0)),
            pl.BlockSpec((ROWS, D), lambda p: (0, 0)),
            pl.BlockSpec((1, ROWS, D), lambda p: (p, 0, 0)),
            pl.BlockSpec((D, D), lambda p: (0, 0)),
            pl.BlockSpec((D, D), lambda p: (0, 0)),
            pl.BlockSpec((1, D), lambda p: (0, 0)),
            pl.BlockSpec((D, 2), lambda p: (0, 0)),
            pl.BlockSpec((1, 2), lambda p: (0, 0)),
            pl.BlockSpec((1, D, PRED), lambda p: (p, 0, 0)),
            pl.BlockSpec((1, PRED), lambda p: (0, 0)),
            pl.BlockSpec((PRED, 2 * PRED), lambda p: (0, 0)),
            pl.BlockSpec((1, 2 * PRED), lambda p: (0, 0)),
            pl.BlockSpec((2 * PRED, PRED), lambda p: (0, 0)),
            pl.BlockSpec((1, PRED), lambda p: (0, 0)),
        ],
        out_specs=pl.BlockSpec((ROWS, PRED), lambda p: (0, 0)),
        out_shape=jax.ShapeDtypeStruct((ROWS, PRED), f32),
        scratch_shapes=[pltpu.VMEM((ROWS, PRED), f32)],
    )(emb3.transpose(1, 0, 2), lm_mean, glob.transpose(1, 0, 2),
      W_g1[:D], W_g1[D:], r2(b_g1), W_g2, r2(b_g2),
      wmh3, r2(b_mh), W_f1, r2(b_f1), W_f2, r2(b_f2))

    out = preds.reshape(B, NV, PRED).transpose(0, 2, 1)
    return out * stdev + means


# speedup vs baseline: 1.5817x; 1.5817x over previous
import math

import jax
import jax.numpy as jnp
from jax.experimental import pallas as pl
from jax.experimental.pallas import tpu as pltpu

B = 32
L = 512
NV = 7
D = 256
PLEN = 16
STRIDE = 8
PADDING = 8
PRED = 96
TOPK = 5
MEM = 100000
NH = 4
HD = D // NH
NP = (L + PADDING - PLEN) // STRIDE + 1
ROWS = B * NV
TILE = 2000
NT = MEM // TILE
NEG = -1e30


def _gelu(x):
    return 0.5 * x * (1.0 + jax.lax.erf(x * 0.7071067811865476))


def _k0(pf_ref, w_ref, b_ref, emb_ref, q_ref):
    e = jax.lax.dot_general(
        pf_ref[...].astype(jnp.bfloat16), w_ref[...].astype(jnp.bfloat16),
        dimension_numbers=(((1,), (0,)), ((), ())),
        precision=jax.lax.Precision.DEFAULT,
        preferred_element_type=jnp.float32) + b_ref[...]
    emb_ref[...] = e
    nq = e.shape[0] // NP
    q_ref[0] = jnp.mean(e.reshape(nq, NP, D), axis=1)


def _k1(q_ref, mem_ref, idx_out_ref, rv_ref, ri_ref):
    t = pl.program_id(0)

    @pl.when(t == 0)
    def _():
        rv_ref[...] = jnp.full((ROWS, TOPK), NEG, jnp.float32)
        ri_ref[...] = jnp.zeros((ROWS, TOPK), jnp.int32)

    sim = jax.lax.dot_general(
        q_ref[...].astype(jnp.bfloat16), mem_ref[...].astype(jnp.bfloat16),
        dimension_numbers=(((1,), (1,)), ((), ())),
        precision=jax.lax.Precision.DEFAULT,
        preferred_element_type=jnp.float32)
    base = t * TILE
    iota_t = jax.lax.broadcasted_iota(jnp.int32, (ROWS, TILE), 1) + base
    aug = jnp.concatenate([sim, rv_ref[...]], axis=1)
    aug_idx = jnp.concatenate([iota_t, ri_ref[...]], axis=1)
    iota_a = jax.lax.broadcasted_iota(jnp.int32, (ROWS, TILE + TOPK), 1)

    new_v = []
    new_i = []
    for _j in range(TOPK):
        m = jnp.max(aug, axis=1, keepdims=True)
        pos = jnp.min(jnp.where(aug == m, iota_a, TILE + TOPK),
                      axis=1, keepdims=True)
        sel = iota_a == pos
        new_v.append(m)
        new_i.append(jnp.sum(jnp.where(sel, aug_idx, 0), axis=1,
                             keepdims=True))
        aug = jnp.where(sel, NEG, aug)
    rv_ref[...] = jnp.concatenate(new_v, axis=1)
    ri_ref[...] = jnp.concatenate(new_i, axis=1)

    @pl.when(t == NT - 1)
    def _():
        idx_out_ref[...] = ri_ref[...]


QB = 8


def _k2(idx_ref, *refs):
    row_refs = refs[:QB * TOPK]
    w1_ref, b1_ref, w2_ref, b2_ref, out_ref = refs[QB * TOPK:]
    rows = jnp.concatenate([r[0] for r in row_refs], axis=0)
    h = _gelu(rows @ w1_ref[...] + b1_ref[...])
    lm = h @ w2_ref[...] + b2_ref[...]
    out_ref[0] = jnp.mean(lm.reshape(QB, TOPK, D), axis=1)


def _k3(x_ref, wq_ref, bq_ref, wk_ref, bk_ref, wv_ref, bv_ref,
        wo_ref, bo_ref, out_ref):
    nb = x_ref.shape[0]
    x2 = x_ref[...].reshape(nb * NP, D)
    q = x2 @ wq_ref[...] + bq_ref[...]
    k = x2 @ wk_ref[...] + bk_ref[...]
    v = x2 @ wv_ref[...] + bv_ref[...]
    scale = 1.0 / math.sqrt(HD)
    outs = []
    for h in range(NH):
        sl = slice(h * HD, (h + 1) * HD)
        qh = q[:, sl].reshape(nb, NP, HD)
        kh = k[:, sl].reshape(nb, NP, HD)
        vh = v[:, sl].reshape(nb, NP, HD)
        s = jax.lax.dot_general(
            qh, kh, dimension_numbers=(((2,), (2,)), ((0,), (0,))),
            preferred_element_type=jnp.float32) * scale
        m = jnp.max(s, axis=-1, keepdims=True)
        e = jnp.exp(s - m)
        a = e / jnp.sum(e, axis=-1, keepdims=True)
        oh = jax.lax.dot_general(
            a, vh, dimension_numbers=(((2,), (1,)), ((0,), (0,))),
            preferred_element_type=jnp.float32)
        outs.append(oh.reshape(nb * NP, HD))
    o = jnp.concatenate(outs, axis=1)
    out_ref[...] = (o @ wo_ref[...] + bo_ref[...]).reshape(nb, NP, D)


PB = 4


def _k4(emb_ref, lm_ref, glob_ref, wg1a_ref, wg1b_ref, bg1_ref,
        wg2_ref, bg2_ref, wmh_ref, bmh_ref, wf1_ref, bf1_ref,
        wf2_ref, bf2_ref, out_ref, acc_ref):
    p = pl.program_id(0)
    loc = (emb_ref[...] + lm_ref[...][None]).reshape(PB * ROWS, D)
    glo = glob_ref[...].reshape(PB * ROWS, D)
    z = _gelu(loc @ wg1a_ref[...] + glo @ wg1b_ref[...] + bg1_ref[...])
    s2 = z @ wg2_ref[...] + bg2_ref[...]
    g0 = jax.nn.sigmoid(s2[:, 0:1] - s2[:, 1:2])
    g1 = 1.0 - g0
    mf = (g0 * loc + g1 * glo).reshape(PB, ROWS, D)
    contrib = mf[0] @ wmh_ref[0]
    for j in range(1, PB):
        contrib = contrib + mf[j] @ wmh_ref[j]

    @pl.when(p == 0)
    def _():
        acc_ref[...] = contrib

    @pl.when(p > 0)
    def _():
        acc_ref[...] = acc_ref[...] + contrib

    @pl.when(p == NP // PB - 1)
    def _():
        mh = acc_ref[...] + bmh_ref[...]
        f = _gelu(_gelu(mh @ wf1_ref[...] + bf1_ref[...])
                  @ wf2_ref[...] + bf2_ref[...])
        out_ref[...] = f + mh


def kernel(x_enc, W_pe, b_pe, mem_bank, W_lm1, b_lm1, W_lm2, b_lm2,
           W_q, b_q, W_k, b_k, W_v, b_v, W_o, b_o,
           W_g1, b_g1, W_g2, b_g2, W_mh, b_mh, W_f1, b_f1, W_f2, b_f2):
    f32 = jnp.float32
    means = jnp.mean(x_enc, axis=1, keepdims=True)
    xc = x_enc - means
    stdev = jnp.sqrt(jnp.var(xc, axis=1, keepdims=True) + 1e-05)
    xn = xc / stdev
    xt = xn.transpose(0, 2, 1)
    xp = jnp.concatenate([xt, jnp.repeat(xt[:, :, -1:], PADDING, axis=-1)],
                         axis=-1)
    uidx = jnp.arange(NP)[:, None] * STRIDE + jnp.arange(PLEN)[None, :]
    patches = xp[:, :, uidx].reshape(ROWS, NP, PLEN)
    pflat = patches.reshape(ROWS * NP, PLEN)
    pmean = patches.mean(axis=1)

    r2 = lambda b: b.reshape(1, -1)

    NB0 = 16
    RB = ROWS * NP // NB0
    emb_flat, q3 = pl.pallas_call(
        _k0,
        grid=(NB0,),
        in_specs=[pl.BlockSpec((RB, PLEN), lambda i: (i, 0)),
                  pl.BlockSpec((PLEN, D), lambda i: (0, 0)),
                  pl.BlockSpec((1, D), lambda i: (0, 0))],
        out_specs=[pl.BlockSpec((RB, D), lambda i: (i, 0)),
                   pl.BlockSpec((1, RB // NP, D), lambda i: (i, 0, 0))],
        out_shape=[jax.ShapeDtypeStruct((ROWS * NP, D), f32),
                   jax.ShapeDtypeStruct((NB0, RB // NP, D), f32)],
    )(pflat, W_pe, r2(b_pe))
    emb3 = emb_flat.reshape(ROWS, NP, D)
    query = q3.reshape(ROWS, D)

    topk_idx = pl.pallas_call(
        _k1,
        grid=(NT,),
        in_specs=[
            pl.BlockSpec((ROWS, D), lambda t: (0, 0)),
            pl.BlockSpec((TILE, D), lambda t: (t, 0)),
        ],
        out_specs=pl.BlockSpec((ROWS, TOPK), lambda t: (0, 0)),
        out_shape=jax.ShapeDtypeStruct((ROWS, TOPK), jnp.int32),
        scratch_shapes=[pltpu.VMEM((ROWS, TOPK), f32),
                        pltpu.VMEM((ROWS, TOPK), jnp.int32)],
    )(query, mem_bank)
    idx_flat = topk_idx.reshape(-1)

    mem3 = mem_bank.reshape(MEM, 1, D)
    nrow = QB * TOPK
    row_spec = [
        pl.BlockSpec((1, 1, D), (lambda i, idx, j=j: (idx[i * nrow + j], 0, 0)))
        for j in range(nrow)
    ]
    lm_mean = pl.pallas_call(
        _k2,
        grid_spec=pltpu.PrefetchScalarGridSpec(
            num_scalar_prefetch=1,
            grid=(ROWS // QB,),
            in_specs=row_spec + [
                pl.BlockSpec((D, 2 * D), lambda i, idx: (0, 0)),
                pl.BlockSpec((1, 2 * D), lambda i, idx: (0, 0)),
                pl.BlockSpec((2 * D, D), lambda i, idx: (0, 0)),
                pl.BlockSpec((1, D), lambda i, idx: (0, 0)),
            ],
            out_specs=pl.BlockSpec((1, QB, D), lambda i, idx: (i, 0, 0)),
        ),
        out_shape=jax.ShapeDtypeStruct((ROWS // QB, QB, D), f32),
    )(idx_flat, *([mem3] * nrow),
      W_lm1, r2(b_lm1), W_lm2, r2(b_lm2))
    lm_mean = lm_mean.reshape(ROWS, D)

    BB = 16
    glob = pl.pallas_call(
        _k3,
        grid=(ROWS // BB,),
        in_specs=[pl.BlockSpec((BB, NP, D), lambda i: (i, 0, 0))] + [
            pl.BlockSpec(s, lambda i: (0, 0))
            for s in [(D, D), (1, D)] * 4
        ],
        out_specs=pl.BlockSpec((BB, NP, D), lambda i: (i, 0, 0)),
        out_shape=jax.ShapeDtypeStruct((ROWS, NP, D), f32),
    )(emb3, W_q, r2(b_q), W_k, r2(b_k), W_v, r2(b_v), W_o, r2(b_o))

    wmh3 = W_mh.reshape(NP, D, PRED)
    preds = pl.pallas_call(
        _k4,
        grid=(NP // PB,),
        in_specs=[
            pl.BlockSpec((PB, ROWS, D), lambda p: (p, 0, 0)),
            pl.BlockSpec((ROWS, D), lambda p: (0, 0)),
            pl.BlockSpec((PB, ROWS, D), lambda p: (p, 0, 0)),
            pl.BlockSpec((D, D), lambda p: (0, 0)),
            pl.BlockSpec((D, D), lambda p: (0, 0)),
            pl.BlockSpec((1, D), lambda p: (0, 0)),
            pl.BlockSpec((D, 2), lambda p: (0, 0)),
            pl.BlockSpec((1, 2), lambda p: (0, 0)),
            pl.BlockSpec((PB, D, PRED), lambda p: (p, 0, 0)),
            pl.BlockSpec((1, PRED), lambda p: (0, 0)),
            pl.BlockSpec((PRED, 2 * PRED), lambda p: (0, 0)),
            pl.BlockSpec((1, 2 * PRED), lambda p: (0, 0)),
            pl.BlockSpec((2 * PRED, PRED), lambda p: (0, 0)),
            pl.BlockSpec((1, PRED), lambda p: (0, 0)),
        ],
        out_specs=pl.BlockSpec((ROWS, PRED), lambda p: (0, 0)),
        out_shape=jax.ShapeDtypeStruct((ROWS, PRED), f32),
        scratch_shapes=[pltpu.VMEM((ROWS, PRED), f32)],
    )(emb3.transpose(1, 0, 2), lm_mean, glob.transpose(1, 0, 2),
      W_g1[:D], W_g1[D:], r2(b_g1), W_g2, r2(b_g2),
      wmh3, r2(b_mh), W_f1, r2(b_f1), W_f2, r2(b_f2))

    out = preds.reshape(B, NV, PRED).transpose(0, 2, 1)
    return out * stdev + means


# speedup vs baseline: 1.6357x; 1.0341x over previous
import math

import jax
import jax.numpy as jnp
from jax.experimental import pallas as pl
from jax.experimental.pallas import tpu as pltpu

B = 32
L = 512
NV = 7
D = 256
PLEN = 16
STRIDE = 8
PADDING = 8
PRED = 96
TOPK = 5
MEM = 100000
NH = 4
HD = D // NH
NP = (L + PADDING - PLEN) // STRIDE + 1
ROWS = B * NV
TILE = 4000
NT = MEM // TILE
NEG = -1e30


def _gelu(x):
    return 0.5 * x * (1.0 + jax.lax.erf(x * 0.7071067811865476))


def _k0(pf_ref, w_ref, b_ref, emb_ref, q_ref):
    e = jax.lax.dot_general(
        pf_ref[...].astype(jnp.bfloat16), w_ref[...].astype(jnp.bfloat16),
        dimension_numbers=(((1,), (0,)), ((), ())),
        precision=jax.lax.Precision.DEFAULT,
        preferred_element_type=jnp.float32) + b_ref[...]
    emb_ref[...] = e
    nq = e.shape[0] // NP
    q_ref[0] = jnp.mean(e.reshape(nq, NP, D), axis=1)


def _k1(q_ref, mem_ref, idx_out_ref, rv_ref, ri_ref):
    t = pl.program_id(0)

    @pl.when(t == 0)
    def _():
        rv_ref[...] = jnp.full((ROWS, TOPK), NEG, jnp.float32)
        ri_ref[...] = jnp.zeros((ROWS, TOPK), jnp.int32)

    sim = jax.lax.dot_general(
        q_ref[...].astype(jnp.bfloat16), mem_ref[...].astype(jnp.bfloat16),
        dimension_numbers=(((1,), (1,)), ((), ())),
        precision=jax.lax.Precision.DEFAULT,
        preferred_element_type=jnp.float32)
    base = t * TILE
    iota_t = jax.lax.broadcasted_iota(jnp.int32, (ROWS, TILE), 1) + base
    aug = jnp.concatenate([sim, rv_ref[...]], axis=1)
    aug_idx = jnp.concatenate([iota_t, ri_ref[...]], axis=1)
    iota_a = jax.lax.broadcasted_iota(jnp.int32, (ROWS, TILE + TOPK), 1)

    new_v = []
    new_i = []
    for _j in range(TOPK):
        m = jnp.max(aug, axis=1, keepdims=True)
        pos = jnp.min(jnp.where(aug == m, iota_a, TILE + TOPK),
                      axis=1, keepdims=True)
        sel = iota_a == pos
        new_v.append(m)
        new_i.append(jnp.sum(jnp.where(sel, aug_idx, 0), axis=1,
                             keepdims=True))
        aug = jnp.where(sel, NEG, aug)
    rv_ref[...] = jnp.concatenate(new_v, axis=1)
    ri_ref[...] = jnp.concatenate(new_i, axis=1)

    @pl.when(t == NT - 1)
    def _():
        idx_out_ref[...] = ri_ref[...]


QB = 16


def _k2(idx_ref, *refs):
    row_refs = refs[:QB * TOPK]
    w1_ref, b1_ref, w2_ref, b2_ref, out_ref = refs[QB * TOPK:]
    rows = jnp.concatenate([r[0] for r in row_refs], axis=0)
    h = _gelu(rows @ w1_ref[...] + b1_ref[...])
    lm = h @ w2_ref[...] + b2_ref[...]
    out_ref[0] = jnp.mean(lm.reshape(QB, TOPK, D), axis=1)


def _k3(x_ref, wq_ref, bq_ref, wk_ref, bk_ref, wv_ref, bv_ref,
        wo_ref, bo_ref, out_ref):
    nb = x_ref.shape[0]
    x2 = x_ref[...].reshape(nb * NP, D)
    q = x2 @ wq_ref[...] + bq_ref[...]
    k = x2 @ wk_ref[...] + bk_ref[...]
    v = x2 @ wv_ref[...] + bv_ref[...]
    scale = 1.0 / math.sqrt(HD)
    outs = []
    for h in range(NH):
        sl = slice(h * HD, (h + 1) * HD)
        qh = q[:, sl].reshape(nb, NP, HD)
        kh = k[:, sl].reshape(nb, NP, HD)
        vh = v[:, sl].reshape(nb, NP, HD)
        s = jax.lax.dot_general(
            qh, kh, dimension_numbers=(((2,), (2,)), ((0,), (0,))),
            preferred_element_type=jnp.float32) * scale
        m = jnp.max(s, axis=-1, keepdims=True)
        e = jnp.exp(s - m)
        a = e / jnp.sum(e, axis=-1, keepdims=True)
        oh = jax.lax.dot_general(
            a, vh, dimension_numbers=(((2,), (1,)), ((0,), (0,))),
            preferred_element_type=jnp.float32)
        outs.append(oh.reshape(nb * NP, HD))
    o = jnp.concatenate(outs, axis=1)
    out_ref[...] = (o @ wo_ref[...] + bo_ref[...]).reshape(nb, NP, D)


PB = 4


def _k4(emb_ref, lm_ref, glob_ref, wg1a_ref, wg1b_ref, bg1_ref,
        wg2_ref, bg2_ref, wmh_ref, bmh_ref, wf1_ref, bf1_ref,
        wf2_ref, bf2_ref, out_ref, acc_ref):
    p = pl.program_id(0)
    loc = (emb_ref[...] + lm_ref[...][None]).reshape(PB * ROWS, D)
    glo = glob_ref[...].reshape(PB * ROWS, D)
    z = _gelu(loc @ wg1a_ref[...] + glo @ wg1b_ref[...] + bg1_ref[...])
    s2 = z @ wg2_ref[...] + bg2_ref[...]
    g0 = jax.nn.sigmoid(s2[:, 0:1] - s2[:, 1:2])
    g1 = 1.0 - g0
    mf = (g0 * loc + g1 * glo).reshape(PB, ROWS, D)
    contrib = mf[0] @ wmh_ref[0]
    for j in range(1, PB):
        contrib = contrib + mf[j] @ wmh_ref[j]

    @pl.when(p == 0)
    def _():
        acc_ref[...] = contrib

    @pl.when(p > 0)
    def _():
        acc_ref[...] = acc_ref[...] + contrib

    @pl.when(p == NP // PB - 1)
    def _():
        mh = acc_ref[...] + bmh_ref[...]
        f = _gelu(_gelu(mh @ wf1_ref[...] + bf1_ref[...])
                  @ wf2_ref[...] + bf2_ref[...])
        out_ref[...] = f + mh


def kernel(x_enc, W_pe, b_pe, mem_bank, W_lm1, b_lm1, W_lm2, b_lm2,
           W_q, b_q, W_k, b_k, W_v, b_v, W_o, b_o,
           W_g1, b_g1, W_g2, b_g2, W_mh, b_mh, W_f1, b_f1, W_f2, b_f2):
    f32 = jnp.float32
    means = jnp.mean(x_enc, axis=1, keepdims=True)
    xc = x_enc - means
    stdev = jnp.sqrt(jnp.var(xc, axis=1, keepdims=True) + 1e-05)
    xn = xc / stdev
    xt = xn.transpose(0, 2, 1)
    xp = jnp.concatenate([xt, jnp.repeat(xt[:, :, -1:], PADDING, axis=-1)],
                         axis=-1)
    uidx = jnp.arange(NP)[:, None] * STRIDE + jnp.arange(PLEN)[None, :]
    patches = xp[:, :, uidx].reshape(ROWS, NP, PLEN)
    pflat = patches.reshape(ROWS * NP, PLEN)
    pmean = patches.mean(axis=1)

    r2 = lambda b: b.reshape(1, -1)

    NB0 = 16
    RB = ROWS * NP // NB0
    emb_flat, q3 = pl.pallas_call(
        _k0,
        grid=(NB0,),
        in_specs=[pl.BlockSpec((RB, PLEN), lambda i: (i, 0)),
                  pl.BlockSpec((PLEN, D), lambda i: (0, 0)),
                  pl.BlockSpec((1, D), lambda i: (0, 0))],
        out_specs=[pl.BlockSpec((RB, D), lambda i: (i, 0)),
                   pl.BlockSpec((1, RB // NP, D), lambda i: (i, 0, 0))],
        out_shape=[jax.ShapeDtypeStruct((ROWS * NP, D), f32),
                   jax.ShapeDtypeStruct((NB0, RB // NP, D), f32)],
    )(pflat, W_pe, r2(b_pe))
    emb3 = emb_flat.reshape(ROWS, NP, D)
    query = q3.reshape(ROWS, D)

    topk_idx = pl.pallas_call(
        _k1,
        grid=(NT,),
        in_specs=[
            pl.BlockSpec((ROWS, D), lambda t: (0, 0)),
            pl.BlockSpec((TILE, D), lambda t: (t, 0)),
        ],
        out_specs=pl.BlockSpec((ROWS, TOPK), lambda t: (0, 0)),
        out_shape=jax.ShapeDtypeStruct((ROWS, TOPK), jnp.int32),
        scratch_shapes=[pltpu.VMEM((ROWS, TOPK), f32),
                        pltpu.VMEM((ROWS, TOPK), jnp.int32)],
    )(query, mem_bank)
    idx_flat = topk_idx.reshape(-1)

    mem3 = mem_bank.reshape(MEM, 1, D)
    nrow = QB * TOPK
    row_spec = [
        pl.BlockSpec((1, 1, D), (lambda i, idx, j=j: (idx[i * nrow + j], 0, 0)))
        for j in range(nrow)
    ]
    lm_mean = pl.pallas_call(
        _k2,
        grid_spec=pltpu.PrefetchScalarGridSpec(
            num_scalar_prefetch=1,
            grid=(ROWS // QB,),
            in_specs=row_spec + [
                pl.BlockSpec((D, 2 * D), lambda i, idx: (0, 0)),
                pl.BlockSpec((1, 2 * D), lambda i, idx: (0, 0)),
                pl.BlockSpec((2 * D, D), lambda i, idx: (0, 0)),
                pl.BlockSpec((1, D), lambda i, idx: (0, 0)),
            ],
            out_specs=pl.BlockSpec((1, QB, D), lambda i, idx: (i, 0, 0)),
        ),
        out_shape=jax.ShapeDtypeStruct((ROWS // QB, QB, D), f32),
    )(idx_flat, *([mem3] * nrow),
      W_lm1, r2(b_lm1), W_lm2, r2(b_lm2))
    lm_mean = lm_mean.reshape(ROWS, D)

    BB = 16
    glob = pl.pallas_call(
        _k3,
        grid=(ROWS // BB,),
        in_specs=[pl.BlockSpec((BB, NP, D), lambda i: (i, 0, 0))] + [
            pl.BlockSpec(s, lambda i: (0, 0))
            for s in [(D, D), (1, D)] * 4
        ],
        out_specs=pl.BlockSpec((BB, NP, D), lambda i: (i, 0, 0)),
        out_shape=jax.ShapeDtypeStruct((ROWS, NP, D), f32),
    )(emb3, W_q, r2(b_q), W_k, r2(b_k), W_v, r2(b_v), W_o, r2(b_o))

    wmh3 = W_mh.reshape(NP, D, PRED)
    preds = pl.pallas_call(
        _k4,
        grid=(NP // PB,),
        in_specs=[
            pl.BlockSpec((PB, ROWS, D), lambda p: (p, 0, 0)),
            pl.BlockSpec((ROWS, D), lambda p: (0, 0)),
            pl.BlockSpec((PB, ROWS, D), lambda p: (p, 0, 0)),
            pl.BlockSpec((D, D), lambda p: (0, 0)),
            pl.BlockSpec((D, D), lambda p: (0, 0)),
            pl.BlockSpec((1, D), lambda p: (0, 0)),
            pl.BlockSpec((D, 2), lambda p: (0, 0)),
            pl.BlockSpec((1, 2), lambda p: (0, 0)),
            pl.BlockSpec((PB, D, PRED), lambda p: (p, 0, 0)),
            pl.BlockSpec((1, PRED), lambda p: (0, 0)),
            pl.BlockSpec((PRED, 2 * PRED), lambda p: (0, 0)),
            pl.BlockSpec((1, 2 * PRED), lambda p: (0, 0)),
            pl.BlockSpec((2 * PRED, PRED), lambda p: (0, 0)),
            pl.BlockSpec((1, PRED), lambda p: (0, 0)),
        ],
        out_specs=pl.BlockSpec((ROWS, PRED), lambda p: (0, 0)),
        out_shape=jax.ShapeDtypeStruct((ROWS, PRED), f32),
        scratch_shapes=[pltpu.VMEM((ROWS, PRED), f32)],
    )(emb3.transpose(1, 0, 2), lm_mean, glob.transpose(1, 0, 2),
      W_g1[:D], W_g1[D:], r2(b_g1), W_g2, r2(b_g2),
      wmh3, r2(b_mh), W_f1, r2(b_f1), W_f2, r2(b_f2))

    out = preds.reshape(B, NV, PRED).transpose(0, 2, 1)
    return out * stdev + means
